# Initial kernel scaffold; baseline (speedup 1.0000x reference)
import jax
import jax.numpy as jnp
from jax.experimental import pallas as pl


def kernel(ent_start_probs, ent_end_probs, ent_part_probs, target_start_probs, target_end_probs, target_part_probs):
    raise NotImplementedError("write your pallas kernel here")


# speedup vs baseline: 4.0102x; 4.0102x over previous
import jax
import jax.numpy as jnp
from jax.experimental import pallas as pl

def _matcher_kernel(ts_ref, te_ref, es_ref, ee_ref, out_ref):
    ws = jnp.exp(ts_ref[0])
    we = jnp.exp(te_ref[0])
    es = es_ref[0]
    ee = ee_ref[0]
    dn = (((1,), (1,)), ((), ()))
    s = jax.lax.dot_general(ws, es, dn, precision=jax.lax.Precision.HIGHEST,
                            preferred_element_type=jnp.float32)
    s = s + jax.lax.dot_general(we, ee, dn, precision=jax.lax.Precision.HIGHEST,
                                preferred_element_type=jnp.float32)
    mx = jnp.max(s, axis=1, keepdims=True)
    iota = jax.lax.broadcasted_iota(jnp.int32, s.shape, 1)
    idx = jnp.min(jnp.where(s == mx, iota, 2**30), axis=1)
    out_ref[0, 0, :] = idx


def kernel(ent_start_probs, ent_end_probs, ent_part_probs,
           target_start_probs, target_end_probs, target_part_probs):
    B, N, L = ent_start_probs.shape
    M = target_start_probs.shape[1]
    out = pl.pallas_call(
        _matcher_kernel,
        grid=(B,),
        in_specs=[
            pl.BlockSpec((1, M, L), lambda i: (i, 0, 0)),
            pl.BlockSpec((1, M, L), lambda i: (i, 0, 0)),
            pl.BlockSpec((1, N, L), lambda i: (i, 0, 0)),
            pl.BlockSpec((1, N, L), lambda i: (i, 0, 0)),
        ],
        out_specs=pl.BlockSpec((1, 1, M), lambda i: (i, 0, 0)),
        out_shape=jax.ShapeDtypeStruct((B, 1, M), jnp.int32),
    )(target_start_probs, target_end_probs, ent_start_probs, ent_end_probs)
    return out.reshape(B, M)
